# baseline (speedup 1.0000x reference)
import functools

import jax
import jax.numpy as jnp
from jax import lax
from jax.experimental import pallas as pl
from jax.experimental.pallas import tpu as pltpu
from jax.experimental.pallas import tpu_sc as plsc

_RC = 8
_L = 16
_RW = 2048


def _sc_kernel_fn(S1, D, B, inT_hbm, tflat_hbm, out_hbm, inT_v, tcol_v,
                  chunk_v, sems):
    NC = 2
    wid = lax.axis_index("s") * NC + lax.axis_index("c")
    nvec = B // _L
    base = wid * _RW
    nrows = _RW + jnp.where(wid == 31, S1 * D - 32 * _RW, 0)
    nchunks = nrows // _RC

    pltpu.sync_copy(inT_hbm, inT_v)
    pltpu.sync_copy(tflat_hbm.at[pl.ds(base, _RW + 64)], tcol_v)

    def chunk_copy(t, buf):
        return pltpu.make_async_copy(
            chunk_v.at[buf],
            out_hbm.at[pl.ds(base + t * _RC, _RC)],
            sems.at[buf])

    def step(t, carry):
        buf = lax.rem(t, 2)

        @pl.when(t >= 2)
        def _wait():
            chunk_copy(t - 2, buf).wait()

        tm = lax.rem(t, D // _RC)
        for j in range(_RC):
            d = tm * _RC + j
            idx = jnp.full((_L,), t * _RC + j, jnp.int32)
            tv = plsc.load_gather(tcol_v, [idx])
            for k in range(nvec):
                sl = pl.ds(k * _L, _L)
                chunk_v[buf, j, sl] = inT_v[d, sl] + tv
        chunk_copy(t, buf).start()
        return carry

    lax.fori_loop(0, nchunks, step, 0)
    chunk_copy(nchunks - 2, lax.rem(nchunks - 2, 2)).wait()
    chunk_copy(nchunks - 1, lax.rem(nchunks - 1, 2)).wait()


def kernel(inputs, table):
    B, _, D = inputs.shape
    S1 = table.shape[0]

    inT = inputs.reshape(B, D).T
    tflat = jnp.pad(table.reshape(-1), (0, 64))

    mesh = plsc.VectorSubcoreMesh(core_axis_name="c", subcore_axis_name="s")
    k = functools.partial(
        pl.kernel,
        mesh=mesh,
        compiler_params=pltpu.CompilerParams(needs_layout_passes=False),
        out_type=jax.ShapeDtypeStruct((S1 * D, B), jnp.float32),
        scratch_types=[
            pltpu.VMEM((D, B), jnp.float32),
            pltpu.VMEM((_RW + 64,), jnp.float32),
            pltpu.VMEM((2, _RC, B), jnp.float32),
            pltpu.SemaphoreType.DMA((2,)),
        ],
    )(functools.partial(_sc_kernel_fn, S1, D, B))
    out2d = k(inT, tflat)
    return out2d.reshape(S1, D, B).transpose(2, 0, 1)


# speedup vs baseline: 9.1709x; 9.1709x over previous
import functools

import jax
import jax.numpy as jnp
from jax.experimental import pallas as pl
from jax.experimental.pallas import tpu as pltpu

_PB = 128
_GP = 8
_NBUF = 8


def _body(nsteps, d, b, inT_ref, tlast_ref, tabT_ref, out_ref, scratch, wbuf,
          sems):
    i = pl.program_id(0)
    lanes = 128
    ltiles = b // lanes

    def group_copy(grp, slot):
        return pltpu.make_async_copy(
            scratch.at[slot], out_ref.at[pl.ds(grp * _GP * d, _GP * d)],
            sems.at[slot])

    for q in range(_PB):
        wbuf[q] = jnp.broadcast_to(tabT_ref[:, q:q + 1], (d, lanes))

    ngroups = _PB // _GP
    for g in range(ngroups):
        slot = g % _NBUF
        grp = i * ngroups + g
        if g < _NBUF:
            @pl.when(i > 0)
            def _wait_prev():
                group_copy(grp - _NBUF, slot).wait()
        else:
            group_copy(grp - _NBUF, slot).wait()
        for j in range(_GP):
            wq = wbuf[g * _GP + j]
            for c in range(ltiles):
                sl = pl.ds(c * lanes, lanes)
                scratch[slot, pl.ds(j * d, d), sl] = inT_ref[:, sl] + wq
        group_copy(grp, slot).start()

    @pl.when(i == nsteps - 1)
    def _tail():
        p_last = nsteps * _PB
        scratch[_NBUF, pl.ds(0, d)] = inT_ref[...] + tlast_ref[...]
        tail_copy = pltpu.make_async_copy(
            scratch.at[_NBUF, pl.ds(0, d)], out_ref.at[pl.ds(p_last * d, d)],
            sems.at[_NBUF])
        tail_copy.start()
        tail_copy.wait()
        for s in range(_NBUF):
            g_last = ngroups - _NBUF + s
            group_copy((nsteps - 1) * ngroups + g_last, s).wait()


def kernel(inputs, table):
    B, _, D = inputs.shape
    S1 = table.shape[0]
    nsteps = (S1 - 1) // _PB
    assert nsteps * _PB == S1 - 1

    inT = inputs.reshape(B, D).T
    tabT = table.T
    tlastT = tabT[:, S1 - 1:S1]

    out2d = pl.pallas_call(
        functools.partial(_body, nsteps, D, B),
        grid=(nsteps,),
        in_specs=[
            pl.BlockSpec(memory_space=pltpu.VMEM),
            pl.BlockSpec(memory_space=pltpu.VMEM),
            pl.BlockSpec((D, _PB), lambda i: (0, i)),
        ],
        out_specs=pl.BlockSpec(memory_space=pl.ANY),
        out_shape=jax.ShapeDtypeStruct((S1 * D, B), jnp.float32),
        scratch_shapes=[
            pltpu.VMEM((_NBUF + 1, _GP * D, B), jnp.float32),
            pltpu.VMEM((_PB, D, 128), jnp.float32),
            pltpu.SemaphoreType.DMA((_NBUF + 1,)),
        ],
    )(inT, tlastT, tabT)
    return out2d.reshape(S1, D, B).transpose(2, 0, 1)
